# baseline (speedup 1.0000x reference)
import functools

import jax
import jax.numpy as jnp
from jax import lax
from jax.experimental import pallas as pl
from jax.experimental.pallas import tpu as pltpu
from jax.experimental.pallas import tpu_sc as plsc

E = 16384
D = 64
_NC = 2
_NS = 16
_NW = _NC * _NS
_CHUNK = 128
_ROWS_PER_W = E // _NW
_CHUNKS_PER_W = _ROWS_PER_W // _CHUNK


def _gather_body(u_tab, v_tab, u_idx, v_idx, u_out, v_out,
                 idx_u, idx_v, urows, vrows, sem):
    wid = lax.axis_index("s") * _NC + lax.axis_index("c")
    rbase = wid * _CHUNKS_PER_W
    pltpu.sync_copy(u_idx.at[pl.ds(rbase, _CHUNKS_PER_W)], idx_u)
    pltpu.sync_copy(v_idx.at[pl.ds(rbase, _CHUNKS_PER_W)], idx_v)
    copies = []
    for j in range(_CHUNKS_PER_W):
        sl = pl.ds(j * _CHUNK, _CHUNK)
        copies.append(pltpu.async_copy(u_tab.at[idx_u.at[j]], urows.at[sl], sem))
        copies.append(pltpu.async_copy(v_tab.at[idx_v.at[j]], vrows.at[sl], sem))
    for c in copies:
        c.wait()
    base = wid * _ROWS_PER_W
    pltpu.sync_copy(urows, u_out.at[pl.ds(base, _ROWS_PER_W)])
    pltpu.sync_copy(vrows, v_out.at[pl.ds(base, _ROWS_PER_W)])


@functools.cache
def _sc_gather():
    return pl.kernel(
        _gather_body,
        out_type=(
            jax.ShapeDtypeStruct((E, D), jnp.float32),
            jax.ShapeDtypeStruct((E, D), jnp.float32),
        ),
        mesh=plsc.VectorSubcoreMesh(core_axis_name="c", subcore_axis_name="s"),
        scratch_types=(
            pltpu.VMEM((_CHUNKS_PER_W, _CHUNK), jnp.int32),
            pltpu.VMEM((_CHUNKS_PER_W, _CHUNK), jnp.int32),
            pltpu.VMEM((_ROWS_PER_W, D), jnp.float32),
            pltpu.VMEM((_ROWS_PER_W, D), jnp.float32),
            pltpu.SemaphoreType.DMA,
        ),
        compiler_params=pltpu.CompilerParams(use_tc_tiling_on_sc=False),
    )


def _compute_body(u_ref, v_ref, w0_ref, w1_ref, w2_ref, ws_ref, out_ref):
    u = u_ref[...]
    v = v_ref[...]
    ws = ws_ref[...]
    logits = None
    for k, w_ref in enumerate((w0_ref, w1_ref, w2_ref)):
        p = jnp.dot(u, w_ref[...], preferred_element_type=jnp.float32)
        x = jnp.sum(p * v, axis=1, keepdims=True)
        contrib = x * ws[k:k + 1, :]
        logits = contrib if logits is None else logits + contrib
    m = jnp.max(logits, axis=1, keepdims=True)
    ex = jnp.exp(logits - m)
    out_ref[...] = ex / jnp.sum(ex, axis=1, keepdims=True)


def _tc_compute(u_g, v_g, W0, W1, W2, weights_scalars, block_e=2048,
                interpret=False):
    grid = (E // block_e,)
    return pl.pallas_call(
        _compute_body,
        grid=grid,
        in_specs=[
            pl.BlockSpec((block_e, D), lambda i: (i, 0)),
            pl.BlockSpec((block_e, D), lambda i: (i, 0)),
            pl.BlockSpec((D, D), lambda i: (0, 0)),
            pl.BlockSpec((D, D), lambda i: (0, 0)),
            pl.BlockSpec((D, D), lambda i: (0, 0)),
            pl.BlockSpec((3, 5), lambda i: (0, 0)),
        ],
        out_specs=pl.BlockSpec((block_e, 5), lambda i: (i, 0)),
        out_shape=jax.ShapeDtypeStruct((E, 5), jnp.float32),
        interpret=interpret,
    )(u_g, v_g, W0, W1, W2, weights_scalars)


def kernel(u_features, v_features, u_indices, v_indices, W0, W1, W2,
           weights_scalars):
    u_idx2 = u_indices.reshape(E // _CHUNK, _CHUNK)
    v_idx2 = v_indices.reshape(E // _CHUNK, _CHUNK)
    u_g, v_g = _sc_gather()(u_features, v_features, u_idx2, v_idx2)
    return _tc_compute(u_g, v_g, W0, W1, W2, weights_scalars)
